# baseline (speedup 1.0000x reference)
import dataclasses
import functools

import jax
import jax.numpy as jnp
from jax import lax
from jax.experimental import pallas as pl
from jax.experimental.pallas import tpu as pltpu
from jax.experimental.pallas import tpu_sc as plsc

_NC = 2
_NS = 16
_NW = _NC * _NS
_L = 16

_N_NODES = 10000
_N_EDGES = 320000
_D = 128

_PER_W = _N_EDGES // _NW
_W = 400
_NSTEP = _PER_W // _W
_SUB = 100
_NSUB = _W // _SUB


def _decode_body(sidx_hbm, tidx_hbm, esgn_hbm, v_hbm, out_hbm,
                 idx_s0, idx_t0, idx_s1, idx_t1,
                 rows_s0, rows_t0, rows_s1, rows_t1,
                 esgn_all, out_all, tbuf,
                 sem_i0, sem_i1, sem_g0, sem_g1, sem_e):
    wid = lax.axis_index("s") * _NC + lax.axis_index("c")
    lane = lax.broadcasted_iota(jnp.int32, (_L,), 0)

    idx_s = [idx_s0, idx_s1]
    idx_t = [idx_t0, idx_t1]
    rows_s = [rows_s0, rows_s1]
    rows_t = [rows_t0, rows_t1]
    sem_i = [sem_i0, sem_i1]
    sem_g = [sem_g0, sem_g1]

    def issue_idx(s, b):
        row0 = wid * (_PER_W // _SUB) + s * _NSUB
        pltpu.async_copy(sidx_hbm.at[pl.ds(row0, _NSUB)], idx_s[b], sem_i[b])
        pltpu.async_copy(tidx_hbm.at[pl.ds(row0, _NSUB)], idx_t[b], sem_i[b])

    def wait_idx(b):
        pltpu.make_async_copy(sidx_hbm.at[pl.ds(0, _NSUB)], idx_s[b],
                              sem_i[b]).wait()
        pltpu.make_async_copy(tidx_hbm.at[pl.ds(0, _NSUB)], idx_t[b],
                              sem_i[b]).wait()

    def issue_gather(b):
        for j in range(_NSUB):
            pltpu.async_copy(v_hbm.at[idx_s[b].at[j]],
                             rows_s[b].at[pl.ds(j * _SUB, _SUB)], sem_g[b])
            pltpu.async_copy(v_hbm.at[idx_t[b].at[j]],
                             rows_t[b].at[pl.ds(j * _SUB, _SUB)], sem_g[b])

    def wait_gather(b):
        pltpu.make_async_copy(v_hbm.at[pl.ds(0, _W)], rows_s[b],
                              sem_g[b]).wait()
        pltpu.make_async_copy(v_hbm.at[pl.ds(0, _W)], rows_t[b],
                              sem_g[b]).wait()

    def compute(s, b):
        rs, rt = rows_s[b], rows_t[b]
        off = s * _W

        @pl.loop(0, _W // _L)
        def _group(g):
            e0 = g * _L
            for e in range(_L):
                row = e0 + e
                parts = []
                for c in range(_D // (2 * _L)):
                    s2 = plsc.bitcast(rs[row, pl.ds(c * _L, _L)],
                                      jnp.bfloat16)
                    t2 = plsc.bitcast(rt[row, pl.ds(c * _L, _L)],
                                      jnp.bfloat16)
                    pa, pb = plsc.unpack(s2 * t2,
                                         format=plsc.PackFormat.INTERLEAVED)
                    parts.append(pa + pb)
                tbuf[e, pl.ds(0, _L)] = ((parts[0] + parts[1]) +
                                         (parts[2] + parts[3]))
            res = plsc.load_gather(tbuf, [lane, jnp.zeros((_L,), jnp.int32)])
            for l in range(1, _L):
                res = res + plsc.load_gather(
                    tbuf, [lane, jnp.full((_L,), l, jnp.int32)])
            out_all[pl.ds(off + e0, _L)] = (res *
                                            esgn_all[pl.ds(off + e0, _L)])

    esgn_copy = pltpu.async_copy(esgn_hbm.at[pl.ds(wid * _PER_W, _PER_W)],
                                 esgn_all, sem_e)
    issue_idx(0, 0)
    wait_idx(0)
    issue_gather(0)
    issue_idx(1, 1)
    esgn_copy.wait()

    @pl.loop(0, _NSTEP - 1, step=2)
    def _chunk2(s0):
        for b in range(2):
            s = s0 + b
            nb = 1 - b

            wait_idx(nb)
            issue_gather(nb)

            wait_gather(b)

            @pl.when(s < _NSTEP - 2)
            def _():
                issue_idx(s + 2, b)

            compute(s, b)

    wait_gather(0)
    compute(_NSTEP - 1, 0)

    pltpu.sync_copy(out_all, out_hbm.at[pl.ds(wid * _PER_W, _PER_W)])


def kernel(v, eidx, esgn):
    sidx = eidx[0].astype(jnp.int32).reshape(_N_EDGES // _SUB, _SUB)
    tidx = eidx[1].astype(jnp.int32).reshape(_N_EDGES // _SUB, _SUB)
    esgn = esgn.astype(jnp.float32)
    v = jax.lax.bitcast_convert_type(
        v.astype(jnp.bfloat16).reshape(_N_NODES, _D // 2, 2), jnp.int32)

    mesh = plsc.VectorSubcoreMesh(core_axis_name="c", subcore_axis_name="s")
    cp = pltpu.CompilerParams()
    if "needs_layout_passes" in pltpu.CompilerParams.__dataclass_fields__:
        cp = dataclasses.replace(cp, needs_layout_passes=False)
    if "use_tc_tiling_on_sc" in pltpu.CompilerParams.__dataclass_fields__:
        cp = dataclasses.replace(cp, use_tc_tiling_on_sc=False)
    run = pl.kernel(
        _decode_body,
        out_type=jax.ShapeDtypeStruct((_N_EDGES,), jnp.float32),
        mesh=mesh,
        scratch_types=[
            pltpu.VMEM((_NSUB, _SUB), jnp.int32),
            pltpu.VMEM((_NSUB, _SUB), jnp.int32),
            pltpu.VMEM((_NSUB, _SUB), jnp.int32),
            pltpu.VMEM((_NSUB, _SUB), jnp.int32),
            pltpu.VMEM((_W, _D // 2), jnp.int32),
            pltpu.VMEM((_W, _D // 2), jnp.int32),
            pltpu.VMEM((_W, _D // 2), jnp.int32),
            pltpu.VMEM((_W, _D // 2), jnp.int32),
            pltpu.VMEM((_PER_W,), jnp.float32),
            pltpu.VMEM((_PER_W,), jnp.float32),
            pltpu.VMEM((_L, 17), jnp.float32),
            pltpu.SemaphoreType.DMA,
            pltpu.SemaphoreType.DMA,
            pltpu.SemaphoreType.DMA,
            pltpu.SemaphoreType.DMA,
            pltpu.SemaphoreType.DMA,
        ],
        compiler_params=cp,
    )
    return run(sidx, tidx, esgn, v)


# speedup vs baseline: 1.0004x; 1.0004x over previous
import dataclasses
import functools

import jax
import jax.numpy as jnp
from jax import lax
from jax.experimental import pallas as pl
from jax.experimental.pallas import tpu as pltpu
from jax.experimental.pallas import tpu_sc as plsc

_NC = 2
_NS = 16
_NW = _NC * _NS
_L = 16

_N_NODES = 10000
_N_EDGES = 320000
_D = 128

_PER_W = _N_EDGES // _NW
_W = 400
_NSTEP = _PER_W // _W
_SUB = 100
_NSUB = _W // _SUB


def _decode_body(sidx_hbm, tidx_hbm, esgn_hbm, v_hbm, out_hbm,
                 idx_s0, idx_t0, idx_s1, idx_t1,
                 rows_s0, rows_t0, rows_s1, rows_t1,
                 esgn_all, out_all, tbuf,
                 sem_i0, sem_i1, sem_g0, sem_g1, sem_e):
    wid = lax.axis_index("s") * _NC + lax.axis_index("c")
    lane = lax.broadcasted_iota(jnp.int32, (_L,), 0)

    idx_s = [idx_s0, idx_s1]
    idx_t = [idx_t0, idx_t1]
    rows_s = [rows_s0, rows_s1]
    rows_t = [rows_t0, rows_t1]
    sem_i = [sem_i0, sem_i1]
    sem_g = [sem_g0, sem_g1]

    def issue_idx(s, b):
        row0 = wid * (_PER_W // _SUB) + s * _NSUB
        pltpu.async_copy(sidx_hbm.at[pl.ds(row0, _NSUB)], idx_s[b], sem_i[b])
        pltpu.async_copy(tidx_hbm.at[pl.ds(row0, _NSUB)], idx_t[b], sem_i[b])

    def wait_idx(b):
        pltpu.make_async_copy(sidx_hbm.at[pl.ds(0, _NSUB)], idx_s[b],
                              sem_i[b]).wait()
        pltpu.make_async_copy(tidx_hbm.at[pl.ds(0, _NSUB)], idx_t[b],
                              sem_i[b]).wait()

    def issue_gather(b):
        for j in range(_NSUB):
            pltpu.async_copy(v_hbm.at[idx_s[b].at[j]],
                             rows_s[b].at[pl.ds(j * _SUB, _SUB)], sem_g[b])
            pltpu.async_copy(v_hbm.at[idx_t[b].at[j]],
                             rows_t[b].at[pl.ds(j * _SUB, _SUB)], sem_g[b])

    def wait_gather(b):
        pltpu.make_async_copy(v_hbm.at[pl.ds(0, _W)], rows_s[b],
                              sem_g[b]).wait()
        pltpu.make_async_copy(v_hbm.at[pl.ds(0, _W)], rows_t[b],
                              sem_g[b]).wait()

    def compute(s, b):
        rs, rt = rows_s[b], rows_t[b]
        off = s * _W

        @pl.loop(0, _W // _L)
        def _group(g):
            e0 = g * _L
            for e in range(_L):
                row = e0 + e
                parts = []
                for c in range(_D // (2 * _L)):
                    s2 = plsc.bitcast(rs[row, pl.ds(c * _L, _L)],
                                      jnp.bfloat16)
                    t2 = plsc.bitcast(rt[row, pl.ds(c * _L, _L)],
                                      jnp.bfloat16)
                    pa, pb = plsc.unpack(s2 * t2,
                                         format=plsc.PackFormat.INTERLEAVED)
                    parts.append(pa + pb)
                tbuf[e, pl.ds(0, _L)] = ((parts[0] + parts[1]) +
                                         (parts[2] + parts[3]))
            res = plsc.load_gather(tbuf, [lane, jnp.zeros((_L,), jnp.int32)])
            for l in range(1, _L):
                res = res + plsc.load_gather(
                    tbuf, [lane, jnp.full((_L,), l, jnp.int32)])
            out_all[pl.ds(off + e0, _L)] = (res *
                                            esgn_all[pl.ds(off + e0, _L)])

    esgn_copy = pltpu.async_copy(esgn_hbm.at[pl.ds(wid * _PER_W, _PER_W)],
                                 esgn_all, sem_e)
    issue_idx(0, 0)
    wait_idx(0)
    issue_gather(0)
    issue_idx(1, 1)
    esgn_copy.wait()

    @pl.loop(0, _NSTEP - 1, step=2)
    def _chunk2(s0):
        for b in range(2):
            s = s0 + b
            nb = 1 - b

            wait_idx(nb)
            issue_gather(nb)

            with jax.named_scope("wait_g"):
                wait_gather(b)

            @pl.when(s < _NSTEP - 2)
            def _():
                issue_idx(s + 2, b)

            with jax.named_scope("comp"):
                compute(s, b)

    wait_gather(0)
    compute(_NSTEP - 1, 0)

    pltpu.sync_copy(out_all, out_hbm.at[pl.ds(wid * _PER_W, _PER_W)])


def kernel(v, eidx, esgn):
    sidx = eidx[0].astype(jnp.int32).reshape(_N_EDGES // _SUB, _SUB)
    tidx = eidx[1].astype(jnp.int32).reshape(_N_EDGES // _SUB, _SUB)
    esgn = esgn.astype(jnp.float32)
    v = jax.lax.bitcast_convert_type(
        v.astype(jnp.bfloat16).reshape(_N_NODES, _D // 2, 2), jnp.int32)

    mesh = plsc.VectorSubcoreMesh(core_axis_name="c", subcore_axis_name="s")
    cp = pltpu.CompilerParams()
    if "needs_layout_passes" in pltpu.CompilerParams.__dataclass_fields__:
        cp = dataclasses.replace(cp, needs_layout_passes=False)
    if "use_tc_tiling_on_sc" in pltpu.CompilerParams.__dataclass_fields__:
        cp = dataclasses.replace(cp, use_tc_tiling_on_sc=False)
    run = pl.kernel(
        _decode_body,
        out_type=jax.ShapeDtypeStruct((_N_EDGES,), jnp.float32),
        mesh=mesh,
        scratch_types=[
            pltpu.VMEM((_NSUB, _SUB), jnp.int32),
            pltpu.VMEM((_NSUB, _SUB), jnp.int32),
            pltpu.VMEM((_NSUB, _SUB), jnp.int32),
            pltpu.VMEM((_NSUB, _SUB), jnp.int32),
            pltpu.VMEM((_W, _D // 2), jnp.int32),
            pltpu.VMEM((_W, _D // 2), jnp.int32),
            pltpu.VMEM((_W, _D // 2), jnp.int32),
            pltpu.VMEM((_W, _D // 2), jnp.int32),
            pltpu.VMEM((_PER_W,), jnp.float32),
            pltpu.VMEM((_PER_W,), jnp.float32),
            pltpu.VMEM((_L, 17), jnp.float32),
            pltpu.SemaphoreType.DMA,
            pltpu.SemaphoreType.DMA,
            pltpu.SemaphoreType.DMA,
            pltpu.SemaphoreType.DMA,
            pltpu.SemaphoreType.DMA,
        ],
        compiler_params=cp,
    )
    return run(sidx, tidx, esgn, v)


# speedup vs baseline: 1.1186x; 1.1182x over previous
import dataclasses
import functools

import jax
import jax.numpy as jnp
from jax import lax
from jax.experimental import pallas as pl
from jax.experimental.pallas import tpu as pltpu
from jax.experimental.pallas import tpu_sc as plsc

_NC = 2
_NS = 16
_NW = _NC * _NS
_L = 16

_N_NODES = 10000
_N_EDGES = 320000
_D = 128

_PER_W = _N_EDGES // _NW
_W = 400
_NSTEP = _PER_W // _W
_SUB = 80
_NSUB = _W // _SUB


def _decode_body(eidx_hbm, esgn_hbm, v_hbm, out_hbm,
                 idx_s0, idx_t0, idx_s1, idx_t1,
                 rows_s0, rows_t0, rows_s1, rows_t1,
                 esgn_all, out_all, tbuf,
                 sem_i0, sem_i1, sem_g0, sem_g1, sem_e):
    wid = lax.axis_index("s") * _NC + lax.axis_index("c")
    lane = lax.broadcasted_iota(jnp.int32, (_L,), 0)

    idx_s = [idx_s0, idx_s1]
    idx_t = [idx_t0, idx_t1]
    rows_s = [rows_s0, rows_s1]
    rows_t = [rows_t0, rows_t1]
    sem_i = [sem_i0, sem_i1]
    sem_g = [sem_g0, sem_g1]

    def issue_idx(s, b):
        base = wid * _PER_W + s * _W
        pltpu.async_copy(eidx_hbm.at[0, pl.ds(base, _W)], idx_s[b], sem_i[b])
        pltpu.async_copy(eidx_hbm.at[1, pl.ds(base, _W)], idx_t[b], sem_i[b])

    def wait_idx(b):
        pltpu.make_async_copy(eidx_hbm.at[0, pl.ds(0, _W)], idx_s[b],
                              sem_i[b]).wait()
        pltpu.make_async_copy(eidx_hbm.at[1, pl.ds(0, _W)], idx_t[b],
                              sem_i[b]).wait()

    def issue_gather(b):
        for j in range(_NSUB):
            pltpu.async_copy(v_hbm.at[idx_s[b].at[pl.ds(j * _SUB, _SUB)]],
                             rows_s[b].at[pl.ds(j * _SUB, _SUB)], sem_g[b])
            pltpu.async_copy(v_hbm.at[idx_t[b].at[pl.ds(j * _SUB, _SUB)]],
                             rows_t[b].at[pl.ds(j * _SUB, _SUB)], sem_g[b])

    def wait_gather(b):
        pltpu.make_async_copy(v_hbm.at[pl.ds(0, _W)], rows_s[b],
                              sem_g[b]).wait()
        pltpu.make_async_copy(v_hbm.at[pl.ds(0, _W)], rows_t[b],
                              sem_g[b]).wait()

    def compute(s, b):
        rs, rt = rows_s[b], rows_t[b]
        off = s * _W

        @pl.loop(0, _W // _L)
        def _group(g):
            e0 = g * _L
            for e in range(_L):
                row = e0 + e
                parts = []
                for c in range(_D // (2 * _L)):
                    s2 = plsc.bitcast(rs[row, pl.ds(c * _L, _L)],
                                      jnp.bfloat16)
                    t2 = plsc.bitcast(rt[row, pl.ds(c * _L, _L)],
                                      jnp.bfloat16)
                    parts.append(s2 * t2)
                q = (parts[0] + parts[1]) + (parts[2] + parts[3])
                qa, qb = plsc.unpack(q, format=plsc.PackFormat.INTERLEAVED)
                tbuf[e, pl.ds(0, _L)] = qa + qb
            res = plsc.load_gather(tbuf, [lane, jnp.zeros((_L,), jnp.int32)])
            for l in range(1, _L):
                res = res + plsc.load_gather(
                    tbuf, [lane, jnp.full((_L,), l, jnp.int32)])
            out_all[pl.ds(off + e0, _L)] = (res *
                                            esgn_all[pl.ds(off + e0, _L)])

    esgn_copy = pltpu.async_copy(esgn_hbm.at[pl.ds(wid * _PER_W, _PER_W)],
                                 esgn_all, sem_e)
    issue_idx(0, 0)
    wait_idx(0)
    issue_gather(0)
    issue_idx(1, 1)
    esgn_copy.wait()

    @pl.loop(0, _NSTEP - 1, step=2)
    def _chunk2(s0):
        for b in range(2):
            s = s0 + b
            nb = 1 - b

            wait_idx(nb)
            issue_gather(nb)

            with jax.named_scope("wait_g"):
                wait_gather(b)

            @pl.when(s < _NSTEP - 2)
            def _():
                issue_idx(s + 2, b)

            with jax.named_scope("comp"):
                compute(s, b)

    wait_gather(0)
    compute(_NSTEP - 1, 0)

    pltpu.sync_copy(out_all, out_hbm.at[pl.ds(wid * _PER_W, _PER_W)])


def kernel(v, eidx, esgn):
    eidx = eidx.astype(jnp.int32)
    esgn = esgn.astype(jnp.float32)
    v = jax.lax.bitcast_convert_type(
        v.astype(jnp.bfloat16).reshape(_N_NODES, _D // 2, 2), jnp.int32)

    mesh = plsc.VectorSubcoreMesh(core_axis_name="c", subcore_axis_name="s")
    cp = pltpu.CompilerParams()
    if "needs_layout_passes" in pltpu.CompilerParams.__dataclass_fields__:
        cp = dataclasses.replace(cp, needs_layout_passes=False)
    if "use_tc_tiling_on_sc" in pltpu.CompilerParams.__dataclass_fields__:
        cp = dataclasses.replace(cp, use_tc_tiling_on_sc=False)
    run = pl.kernel(
        _decode_body,
        out_type=jax.ShapeDtypeStruct((_N_EDGES,), jnp.float32),
        mesh=mesh,
        scratch_types=[
            pltpu.VMEM((_W,), jnp.int32),
            pltpu.VMEM((_W,), jnp.int32),
            pltpu.VMEM((_W,), jnp.int32),
            pltpu.VMEM((_W,), jnp.int32),
            pltpu.VMEM((_W, _D // 2), jnp.int32),
            pltpu.VMEM((_W, _D // 2), jnp.int32),
            pltpu.VMEM((_W, _D // 2), jnp.int32),
            pltpu.VMEM((_W, _D // 2), jnp.int32),
            pltpu.VMEM((_PER_W,), jnp.float32),
            pltpu.VMEM((_PER_W,), jnp.float32),
            pltpu.VMEM((_L, 17), jnp.float32),
            pltpu.SemaphoreType.DMA,
            pltpu.SemaphoreType.DMA,
            pltpu.SemaphoreType.DMA,
            pltpu.SemaphoreType.DMA,
            pltpu.SemaphoreType.DMA,
        ],
        compiler_params=cp,
    )
    return run(eidx, esgn, v)
